# baseline (speedup 1.0000x reference)
import functools

import jax
import jax.numpy as jnp
from jax import lax
from jax.experimental import pallas as pl
from jax.experimental.pallas import tpu as pltpu
from jax.experimental.pallas import tpu_sc as plsc

N = 10000
E = 320000
IN_CH = 128
HID = 16

NC, NS = 2, 16
NW = NC * NS
EPW = E // NW

CHUNK = 2048
NFULL = 4
TAIL = EPW - NFULL * CHUNK

NPAD = 10112
ROWS_PER_TILE = NPAD // NS


def _tc_transform(x_ref, wl_ref, wr_ref, y1_ref, xr_ref):
    xx = x_ref[...]
    y1_ref[...] = lax.dot(xx, wl_ref[...], precision=lax.Precision.HIGHEST,
                          preferred_element_type=jnp.float32)
    xr_ref[...] = lax.dot(xx, wr_ref[...], precision=lax.Precision.HIGHEST,
                          preferred_element_type=jnp.float32)


_ROWS_BLK = 1000

_transform_call = pl.pallas_call(
    _tc_transform,
    grid=(N // _ROWS_BLK,),
    in_specs=[
        pl.BlockSpec((_ROWS_BLK, IN_CH), lambda i: (i, 0)),
        pl.BlockSpec((IN_CH, HID), lambda i: (0, 0)),
        pl.BlockSpec((IN_CH, HID), lambda i: (0, 0)),
    ],
    out_specs=(pl.BlockSpec((_ROWS_BLK, HID), lambda i: (i, 0)),
               pl.BlockSpec((_ROWS_BLK, HID), lambda i: (i, 0))),
    out_shape=(jax.ShapeDtypeStruct((N, HID), jnp.float32),
               jax.ShapeDtypeStruct((N, HID), jnp.float32)),
)


def _sc_layer1(y1_hbm, srcf_hbm, dstf_hbm, agg_out, cnt_out,
               src_c, dst_c, src_t, dst_t, rows_v, zrow_v, cnt_v, y1_sh, acc_sh):
    cid = lax.axis_index("c")
    sid = lax.axis_index("s")
    wid = cid * NS + sid

    @pl.when(sid == 0)
    def _():
        pltpu.sync_copy(y1_hbm, y1_sh)

    zeros16 = jnp.zeros((16,), jnp.float32)

    def zb(i, carry):
        zrow_v[i, :] = zeros16
        cnt_v[pl.ds(i * 16, 16)] = zeros16
        return carry

    lax.fori_loop(0, ROWS_PER_TILE, zb, 0)
    pltpu.sync_copy(zrow_v, acc_sh.at[pl.ds(sid * ROWS_PER_TILE, ROWS_PER_TILE), :])

    plsc.subcore_barrier()

    ones16 = jnp.full((16,), 1.0, jnp.float32)

    def do_chunk(idx_ref_s, idx_ref_d, rows_slice, size, base):
        pltpu.sync_copy(srcf_hbm.at[pl.ds(base, size)], idx_ref_s)
        pltpu.sync_copy(dstf_hbm.at[pl.ds(base, size)], idx_ref_d)
        pltpu.sync_copy(y1_sh.at[idx_ref_s], rows_slice)
        pltpu.sync_copy(rows_slice, acc_sh.at[idx_ref_d], add=True)

        def cnt_body(i, c2):
            d16 = idx_ref_d[pl.ds(i * 16, 16)]
            plsc.addupdate_scatter(cnt_v, [d16], ones16)
            return c2

        lax.fori_loop(0, size // 16, cnt_body, 0)

    def chunk(j, carry):
        do_chunk(src_c, dst_c, rows_v, CHUNK, wid * EPW + j * CHUNK)
        return carry

    lax.fori_loop(0, NFULL, chunk, 0)
    do_chunk(src_t, dst_t, rows_v.at[pl.ds(0, TAIL), :], TAIL,
             wid * EPW + NFULL * CHUNK)

    plsc.subcore_barrier()
    pltpu.sync_copy(acc_sh.at[pl.ds(sid * ROWS_PER_TILE, ROWS_PER_TILE), :],
                    agg_out.at[cid, pl.ds(sid * ROWS_PER_TILE, ROWS_PER_TILE), :])
    pltpu.sync_copy(cnt_v.at[pl.ds(0, N)], cnt_out.at[pl.ds(wid * N, N)])


_layer1_call = pl.kernel(
    _sc_layer1,
    out_type=(jax.ShapeDtypeStruct((NC, NPAD, HID), jnp.float32),
              jax.ShapeDtypeStruct((NW * N,), jnp.float32)),
    mesh=plsc.VectorSubcoreMesh(core_axis_name="c", subcore_axis_name="s",
                                num_cores=NC, num_subcores=NS),
    compiler_params=pltpu.CompilerParams(needs_layout_passes=False, use_tc_tiling_on_sc=False),
    scratch_types=[
        pltpu.VMEM((CHUNK,), jnp.int32),
        pltpu.VMEM((CHUNK,), jnp.int32),
        pltpu.VMEM((TAIL,), jnp.int32),
        pltpu.VMEM((TAIL,), jnp.int32),
        pltpu.VMEM((CHUNK, HID), jnp.float32),
        pltpu.VMEM((ROWS_PER_TILE, HID), jnp.float32),
        pltpu.VMEM((NPAD,), jnp.float32),
        pltpu.VMEM_SHARED((N, HID), jnp.float32),
        pltpu.VMEM_SHARED((NPAD, HID), jnp.float32),
    ],
)


def _tc_mid(agg_ref, cntp_ref, xr_ref, b1_ref, w2l_ref, w2r_ref, b2_ref,
            y2_ref, base2_ref, c_ref):
    cnt = cntp_ref[pl.ds(0, N)]
    for w in range(1, NW):
        cnt = cnt + cntp_ref[pl.ds(w * N, N)]
    c = jnp.maximum(cnt, 1.0)
    agg = (agg_ref[0] + agg_ref[1])[:N, :]
    h = jnp.maximum(agg / c[:, None] + xr_ref[...] + b1_ref[...][None, :], 0.0)
    w2l = w2l_ref[...][:, 0]
    w2r = w2r_ref[...][:, 0]
    y2_ref[...] = jnp.sum(h * w2l[None, :], axis=1)
    base2_ref[...] = jnp.sum(h * w2r[None, :], axis=1) + b2_ref[...][0]
    c_ref[...] = c


_mid_call = pl.pallas_call(
    _tc_mid,
    out_shape=(jax.ShapeDtypeStruct((N,), jnp.float32),
               jax.ShapeDtypeStruct((N,), jnp.float32),
               jax.ShapeDtypeStruct((N,), jnp.float32)),
)


def _sc_layer2(y2_hbm, srcf_hbm, dstf_hbm, out_hbm, y2_v, srcf_v, dstf_v, acc_v):
    cid = lax.axis_index("c")
    sid = lax.axis_index("s")
    wid = cid * NS + sid

    zeros16 = jnp.zeros((16,), jnp.float32)

    def zb(i, carry):
        acc_v[pl.ds(i * 16, 16)] = zeros16
        return carry

    lax.fori_loop(0, N // 16, zb, 0)

    pltpu.sync_copy(y2_hbm, y2_v)
    pltpu.sync_copy(srcf_hbm.at[pl.ds(wid * EPW, EPW)], srcf_v)
    pltpu.sync_copy(dstf_hbm.at[pl.ds(wid * EPW, EPW)], dstf_v)

    def step(i, carry):
        s16 = srcf_v[pl.ds(i * 16, 16)]
        d16 = dstf_v[pl.ds(i * 16, 16)]
        vals = plsc.load_gather(y2_v, [s16])
        plsc.addupdate_scatter(acc_v, [d16], vals)
        return carry

    lax.fori_loop(0, EPW // 16, step, 0)

    pltpu.sync_copy(acc_v.at[pl.ds(0, N)], out_hbm.at[pl.ds(wid * N, N)])


_layer2_call = pl.kernel(
    _sc_layer2,
    out_type=jax.ShapeDtypeStruct((NW * N,), jnp.float32),
    mesh=plsc.VectorSubcoreMesh(core_axis_name="c", subcore_axis_name="s",
                                num_cores=NC, num_subcores=NS),
    compiler_params=pltpu.CompilerParams(needs_layout_passes=False, use_tc_tiling_on_sc=False),
    scratch_types=[
        pltpu.VMEM((N,), jnp.float32),
        pltpu.VMEM((EPW,), jnp.int32),
        pltpu.VMEM((EPW,), jnp.int32),
        pltpu.VMEM((N,), jnp.float32),
    ],
)


def _tc_final(aggp_ref, c_ref, base2_ref, out_ref):
    s = aggp_ref[pl.ds(0, N)]
    for w in range(1, NW):
        s = s + aggp_ref[pl.ds(w * N, N)]
    out_ref[...] = s / c_ref[...] + base2_ref[...]


_final_call = pl.pallas_call(
    _tc_final,
    out_shape=jax.ShapeDtypeStruct((N,), jnp.float32),
)


def kernel(x, edge_index, W1_l, W1_r, b1, W2_l, W2_r, b2):
    ei = edge_index.astype(jnp.int32)
    src = ei[0]
    dst = ei[1]
    y1, xr = _transform_call(x, W1_l, W1_r)
    agg_p, cnt_p = _layer1_call(y1, src, dst)
    y2, base2, c = _mid_call(agg_p, cnt_p, xr, b1, W2_l, W2_r, b2)
    agg2_p = _layer2_call(y2, src, dst)
    return _final_call(agg2_p, c, base2)


# speedup vs baseline: 1.1318x; 1.1318x over previous
import functools

import jax
import jax.numpy as jnp
from jax import lax
from jax.experimental import pallas as pl
from jax.experimental.pallas import tpu as pltpu
from jax.experimental.pallas import tpu_sc as plsc

N = 10000
E = 320000
IN_CH = 128
HID = 16

NC, NS = 2, 16
NW = NC * NS
EPW = E // NW

CHUNK = 2048
NFULL = 4
TAIL = EPW - NFULL * CHUNK

NPAD = 10112
ROWS_PER_TILE = NPAD // NS


def _tc_transform(x_ref, wl_ref, wr_ref, y1_ref, xr_ref):
    xx = x_ref[...]
    y1_ref[...] = lax.dot(xx, wl_ref[...], precision=lax.Precision.HIGHEST,
                          preferred_element_type=jnp.float32)
    xr_ref[...] = lax.dot(xx, wr_ref[...], precision=lax.Precision.HIGHEST,
                          preferred_element_type=jnp.float32)


_transform_call = pl.pallas_call(
    _tc_transform,
    out_shape=(jax.ShapeDtypeStruct((N, HID), jnp.float32),
               jax.ShapeDtypeStruct((N, HID), jnp.float32)),
)


def _sc_layer1(y1_hbm, edge_hbm, agg_out, cnt_out,
               src_c, dst_c, src_t, dst_t, rows_v, zrow_v, cnt_v, y1_sh, acc_sh):
    cid = lax.axis_index("c")
    sid = lax.axis_index("s")
    wid = cid * NS + sid

    @pl.when(sid == 0)
    def _():
        pltpu.sync_copy(y1_hbm, y1_sh)

    zeros16 = jnp.zeros((16,), jnp.float32)

    def zb(i, carry):
        zrow_v[i, :] = zeros16
        cnt_v[pl.ds(i * 16, 16)] = zeros16
        return carry

    lax.fori_loop(0, ROWS_PER_TILE, zb, 0)
    pltpu.sync_copy(zrow_v, acc_sh.at[pl.ds(sid * ROWS_PER_TILE, ROWS_PER_TILE), :])

    plsc.subcore_barrier()

    ones16 = jnp.full((16,), 1.0, jnp.float32)

    def do_chunk(idx_ref_s, idx_ref_d, rows_slice, size, base):
        pltpu.sync_copy(edge_hbm.at[0, pl.ds(base, size)], idx_ref_s)
        pltpu.sync_copy(edge_hbm.at[1, pl.ds(base, size)], idx_ref_d)
        pltpu.sync_copy(y1_sh.at[idx_ref_s], rows_slice)
        pltpu.sync_copy(rows_slice, acc_sh.at[idx_ref_d], add=True)

        def cnt_body(i, c2):
            d16 = idx_ref_d[pl.ds(i * 16, 16)]
            plsc.addupdate_scatter(cnt_v, [d16], ones16)
            return c2

        lax.fori_loop(0, size // 16, cnt_body, 0)

    def chunk(j, carry):
        do_chunk(src_c, dst_c, rows_v, CHUNK, wid * EPW + j * CHUNK)
        return carry

    lax.fori_loop(0, NFULL, chunk, 0)
    do_chunk(src_t, dst_t, rows_v.at[pl.ds(0, TAIL), :], TAIL,
             wid * EPW + NFULL * CHUNK)

    plsc.subcore_barrier()
    pltpu.sync_copy(acc_sh.at[pl.ds(sid * ROWS_PER_TILE, ROWS_PER_TILE), :],
                    agg_out.at[cid, pl.ds(sid * ROWS_PER_TILE, ROWS_PER_TILE), :])
    pltpu.sync_copy(cnt_v.at[pl.ds(0, N)], cnt_out.at[pl.ds(wid * N, N)])


_layer1_call = pl.kernel(
    _sc_layer1,
    out_type=(jax.ShapeDtypeStruct((NC, NPAD, HID), jnp.float32),
              jax.ShapeDtypeStruct((NW * N,), jnp.float32)),
    mesh=plsc.VectorSubcoreMesh(core_axis_name="c", subcore_axis_name="s",
                                num_cores=NC, num_subcores=NS),
    compiler_params=pltpu.CompilerParams(needs_layout_passes=False, use_tc_tiling_on_sc=False),
    scratch_types=[
        pltpu.VMEM((CHUNK,), jnp.int32),
        pltpu.VMEM((CHUNK,), jnp.int32),
        pltpu.VMEM((TAIL,), jnp.int32),
        pltpu.VMEM((TAIL,), jnp.int32),
        pltpu.VMEM((CHUNK, HID), jnp.float32),
        pltpu.VMEM((ROWS_PER_TILE, HID), jnp.float32),
        pltpu.VMEM((NPAD,), jnp.float32),
        pltpu.VMEM_SHARED((N, HID), jnp.float32),
        pltpu.VMEM_SHARED((NPAD, HID), jnp.float32),
    ],
)


def _tc_mid(agg_ref, cntp_ref, xr_ref, b1_ref, w2l_ref, w2r_ref, b2_ref,
            y2_ref, base2_ref, c_ref):
    cnt = cntp_ref[pl.ds(0, N)]
    for w in range(1, NW):
        cnt = cnt + cntp_ref[pl.ds(w * N, N)]
    c = jnp.maximum(cnt, 1.0)
    agg = (agg_ref[0] + agg_ref[1])[:N, :]
    h = jnp.maximum(agg / c[:, None] + xr_ref[...] + b1_ref[...][None, :], 0.0)
    w2l = w2l_ref[...][:, 0]
    w2r = w2r_ref[...][:, 0]
    y2_ref[...] = jnp.sum(h * w2l[None, :], axis=1)
    base2_ref[...] = jnp.sum(h * w2r[None, :], axis=1) + b2_ref[...][0]
    c_ref[...] = c


_mid_call = pl.pallas_call(
    _tc_mid,
    out_shape=(jax.ShapeDtypeStruct((N,), jnp.float32),
               jax.ShapeDtypeStruct((N,), jnp.float32),
               jax.ShapeDtypeStruct((N,), jnp.float32)),
)


def _sc_layer2(y2_hbm, edge_hbm, out_hbm, y2_v, srcf_v, dstf_v, acc_v):
    cid = lax.axis_index("c")
    sid = lax.axis_index("s")
    wid = cid * NS + sid

    zeros16 = jnp.zeros((16,), jnp.float32)

    def zb(i, carry):
        acc_v[pl.ds(i * 16, 16)] = zeros16
        return carry

    lax.fori_loop(0, N // 16, zb, 0)

    pltpu.sync_copy(y2_hbm, y2_v)
    pltpu.sync_copy(edge_hbm.at[0, pl.ds(wid * EPW, EPW)], srcf_v)
    pltpu.sync_copy(edge_hbm.at[1, pl.ds(wid * EPW, EPW)], dstf_v)

    def step(i, carry):
        s16 = srcf_v[pl.ds(i * 16, 16)]
        d16 = dstf_v[pl.ds(i * 16, 16)]
        vals = plsc.load_gather(y2_v, [s16])
        plsc.addupdate_scatter(acc_v, [d16], vals)
        return carry

    lax.fori_loop(0, EPW // 16, step, 0)

    pltpu.sync_copy(acc_v.at[pl.ds(0, N)], out_hbm.at[pl.ds(wid * N, N)])


_layer2_call = pl.kernel(
    _sc_layer2,
    out_type=jax.ShapeDtypeStruct((NW * N,), jnp.float32),
    mesh=plsc.VectorSubcoreMesh(core_axis_name="c", subcore_axis_name="s",
                                num_cores=NC, num_subcores=NS),
    compiler_params=pltpu.CompilerParams(needs_layout_passes=False, use_tc_tiling_on_sc=False),
    scratch_types=[
        pltpu.VMEM((N,), jnp.float32),
        pltpu.VMEM((EPW,), jnp.int32),
        pltpu.VMEM((EPW,), jnp.int32),
        pltpu.VMEM((N,), jnp.float32),
    ],
)


def _tc_final(aggp_ref, c_ref, base2_ref, out_ref):
    s = aggp_ref[pl.ds(0, N)]
    for w in range(1, NW):
        s = s + aggp_ref[pl.ds(w * N, N)]
    out_ref[...] = s / c_ref[...] + base2_ref[...]


_final_call = pl.pallas_call(
    _tc_final,
    out_shape=jax.ShapeDtypeStruct((N,), jnp.float32),
)


def kernel(x, edge_index, W1_l, W1_r, b1, W2_l, W2_r, b2):
    ei = edge_index.astype(jnp.int32)
    y1, xr = _transform_call(x, W1_l, W1_r)
    agg_p, cnt_p = _layer1_call(y1, ei)
    y2, base2, c = _mid_call(agg_p, cnt_p, xr, b1, W2_l, W2_r, b2)
    agg2_p = _layer2_call(y2, ei)
    return _final_call(agg2_p, c, base2)


# speedup vs baseline: 1.1455x; 1.0121x over previous
import functools

import jax
import jax.numpy as jnp
from jax import lax
from jax.experimental import pallas as pl
from jax.experimental.pallas import tpu as pltpu
from jax.experimental.pallas import tpu_sc as plsc

N = 10000
E = 320000
IN_CH = 128
HID = 16

NC, NS = 2, 16
NW = NC * NS
EPW = E // NW

CHUNK = 2048
NFULL = 4
TAIL = EPW - NFULL * CHUNK

NPAD = 10112
ROWS_PER_TILE = NPAD // NS


def _tc_transform(x_ref, wcat_ref, yx_ref):
    yx_ref[...] = lax.dot(x_ref[...], wcat_ref[...],
                          precision=lax.Precision.HIGHEST,
                          preferred_element_type=jnp.float32)


_transform_call = pl.pallas_call(
    _tc_transform,
    out_shape=jax.ShapeDtypeStruct((N, IN_CH), jnp.float32),
)


def _sc_layer1(yx_hbm, edge_hbm, agg_out, cnt_out,
               src_c, dst_c, src_t, dst_t, rows_v, zrow_v, cnt_v, y1_sh, acc_sh):
    cid = lax.axis_index("c")
    sid = lax.axis_index("s")
    wid = cid * NS + sid

    _L1ROWS = 10000 - 15 * ROWS_PER_TILE

    @pl.when(sid < 15)
    def _():
        r0 = sid * ROWS_PER_TILE
        pltpu.sync_copy(yx_hbm.at[pl.ds(r0, ROWS_PER_TILE), pl.ds(0, HID)],
                        y1_sh.at[pl.ds(r0, ROWS_PER_TILE), :])

    @pl.when(sid == 15)
    def _():
        r0 = 15 * ROWS_PER_TILE
        pltpu.sync_copy(yx_hbm.at[pl.ds(r0, _L1ROWS), pl.ds(0, HID)],
                        y1_sh.at[pl.ds(r0, _L1ROWS), :])

    zeros16 = jnp.zeros((16,), jnp.float32)

    def zb(i, carry):
        zrow_v[i, :] = zeros16
        cnt_v[pl.ds(i * 16, 16)] = zeros16
        return carry

    lax.fori_loop(0, ROWS_PER_TILE, zb, 0)
    pltpu.sync_copy(zrow_v, acc_sh.at[pl.ds(sid * ROWS_PER_TILE, ROWS_PER_TILE), :])

    plsc.subcore_barrier()

    ones16 = jnp.full((16,), 1.0, jnp.float32)

    def do_chunk(idx_ref_s, idx_ref_d, rows_slice, size, base):
        pltpu.sync_copy(edge_hbm.at[0, pl.ds(base, size)], idx_ref_s)
        pltpu.sync_copy(edge_hbm.at[1, pl.ds(base, size)], idx_ref_d)
        pltpu.sync_copy(y1_sh.at[idx_ref_s], rows_slice)
        pltpu.sync_copy(rows_slice, acc_sh.at[idx_ref_d], add=True)

        def cnt_body(i, c2):
            d16 = idx_ref_d[pl.ds(i * 16, 16)]
            plsc.addupdate_scatter(cnt_v, [d16], ones16)
            return c2

        lax.fori_loop(0, size // 16, cnt_body, 0)

    def chunk(j, carry):
        do_chunk(src_c, dst_c, rows_v, CHUNK, wid * EPW + j * CHUNK)
        return carry

    lax.fori_loop(0, NFULL, chunk, 0)
    do_chunk(src_t, dst_t, rows_v.at[pl.ds(0, TAIL), :], TAIL,
             wid * EPW + NFULL * CHUNK)

    plsc.subcore_barrier()
    pltpu.sync_copy(acc_sh.at[pl.ds(sid * ROWS_PER_TILE, ROWS_PER_TILE), :],
                    agg_out.at[cid, pl.ds(sid * ROWS_PER_TILE, ROWS_PER_TILE), :])
    pltpu.sync_copy(cnt_v.at[pl.ds(0, N)], cnt_out.at[pl.ds(wid * N, N)])


_layer1_call = pl.kernel(
    _sc_layer1,
    out_type=(jax.ShapeDtypeStruct((NC, NPAD, HID), jnp.float32),
              jax.ShapeDtypeStruct((NW * N,), jnp.float32)),
    mesh=plsc.VectorSubcoreMesh(core_axis_name="c", subcore_axis_name="s",
                                num_cores=NC, num_subcores=NS),
    compiler_params=pltpu.CompilerParams(needs_layout_passes=False, use_tc_tiling_on_sc=False),
    scratch_types=[
        pltpu.VMEM((CHUNK,), jnp.int32),
        pltpu.VMEM((CHUNK,), jnp.int32),
        pltpu.VMEM((TAIL,), jnp.int32),
        pltpu.VMEM((TAIL,), jnp.int32),
        pltpu.VMEM((CHUNK, HID), jnp.float32),
        pltpu.VMEM((ROWS_PER_TILE, HID), jnp.float32),
        pltpu.VMEM((NPAD,), jnp.float32),
        pltpu.VMEM_SHARED((N, HID), jnp.float32),
        pltpu.VMEM_SHARED((NPAD, HID), jnp.float32),
    ],
)


def _tc_mid(agg_ref, cntp_ref, yx_ref, b1_ref, w2l_ref, w2r_ref, b2_ref,
            y2_ref, base2_ref, c_ref):
    cnt = cntp_ref[pl.ds(0, N)]
    for w in range(1, NW):
        cnt = cnt + cntp_ref[pl.ds(w * N, N)]
    c = jnp.maximum(cnt, 1.0)
    agg = (agg_ref[0] + agg_ref[1])[:N, :]
    xr = yx_ref[:, HID:2 * HID]
    h = jnp.maximum(agg / c[:, None] + xr + b1_ref[...][None, :], 0.0)
    w2l = w2l_ref[...][:, 0]
    w2r = w2r_ref[...][:, 0]
    y2_ref[...] = jnp.sum(h * w2l[None, :], axis=1)
    base2_ref[...] = jnp.sum(h * w2r[None, :], axis=1) + b2_ref[...][0]
    c_ref[...] = c


_mid_call = pl.pallas_call(
    _tc_mid,
    out_shape=(jax.ShapeDtypeStruct((N,), jnp.float32),
               jax.ShapeDtypeStruct((N,), jnp.float32),
               jax.ShapeDtypeStruct((N,), jnp.float32)),
)


def _sc_layer2(y2_hbm, edge_hbm, out_hbm, y2_v, srcf_v, dstf_v, acc_v):
    cid = lax.axis_index("c")
    sid = lax.axis_index("s")
    wid = cid * NS + sid

    zeros16 = jnp.zeros((16,), jnp.float32)

    def zb(i, carry):
        acc_v[pl.ds(i * 16, 16)] = zeros16
        return carry

    lax.fori_loop(0, N // 16, zb, 0)

    pltpu.sync_copy(y2_hbm, y2_v)
    pltpu.sync_copy(edge_hbm.at[0, pl.ds(wid * EPW, EPW)], srcf_v)
    pltpu.sync_copy(edge_hbm.at[1, pl.ds(wid * EPW, EPW)], dstf_v)

    def step(i, carry):
        s16 = srcf_v[pl.ds(i * 16, 16)]
        d16 = dstf_v[pl.ds(i * 16, 16)]
        vals = plsc.load_gather(y2_v, [s16])
        plsc.addupdate_scatter(acc_v, [d16], vals)
        return carry

    lax.fori_loop(0, EPW // 16, step, 0)

    pltpu.sync_copy(acc_v.at[pl.ds(0, N)], out_hbm.at[pl.ds(wid * N, N)])


_layer2_call = pl.kernel(
    _sc_layer2,
    out_type=jax.ShapeDtypeStruct((NW * N,), jnp.float32),
    mesh=plsc.VectorSubcoreMesh(core_axis_name="c", subcore_axis_name="s",
                                num_cores=NC, num_subcores=NS),
    compiler_params=pltpu.CompilerParams(needs_layout_passes=False, use_tc_tiling_on_sc=False),
    scratch_types=[
        pltpu.VMEM((N,), jnp.float32),
        pltpu.VMEM((EPW,), jnp.int32),
        pltpu.VMEM((EPW,), jnp.int32),
        pltpu.VMEM((N,), jnp.float32),
    ],
)


def _tc_final(aggp_ref, c_ref, base2_ref, out_ref):
    s = aggp_ref[pl.ds(0, N)]
    for w in range(1, NW):
        s = s + aggp_ref[pl.ds(w * N, N)]
    out_ref[...] = s / c_ref[...] + base2_ref[...]


_final_call = pl.pallas_call(
    _tc_final,
    out_shape=jax.ShapeDtypeStruct((N,), jnp.float32),
)


def kernel(x, edge_index, W1_l, W1_r, b1, W2_l, W2_r, b2):
    ei = edge_index.astype(jnp.int32)
    wcat = jnp.zeros((IN_CH, IN_CH), jnp.float32)
    wcat = wcat.at[:, :HID].set(W1_l).at[:, HID:2 * HID].set(W1_r)
    yx = _transform_call(x, wcat)
    agg_p, cnt_p = _layer1_call(yx, ei)
    y2, base2, c = _mid_call(agg_p, cnt_p, yx, b1, W2_l, W2_r, b2)
    agg2_p = _layer2_call(y2, ei)
    return _final_call(agg2_p, c, base2)


# speedup vs baseline: 1.2773x; 1.1151x over previous
import functools

import jax
import jax.numpy as jnp
from jax import lax
from jax.experimental import pallas as pl
from jax.experimental.pallas import tpu as pltpu
from jax.experimental.pallas import tpu_sc as plsc

N = 10000
E = 320000
IN_CH = 128
HID = 16

NC, NS = 2, 16
NW = NC * NS
EPW = E // NW

CHUNK = 2048
NFULL = 4
TAIL = EPW - NFULL * CHUNK

NPAD = 10240
ROWS_PER_TILE = NPAD // NS
NGRP = ROWS_PER_TILE // 16
LAST_ROWS = N - 15 * ROWS_PER_TILE
LAST_GRP = LAST_ROWS // 16


def _tc_transform(x_ref, wcat_ref, yx_ref):
    yx_ref[...] = lax.dot(x_ref[...], wcat_ref[...],
                          precision=lax.Precision.HIGHEST,
                          preferred_element_type=jnp.float32)


_transform_call = pl.pallas_call(
    _tc_transform,
    out_shape=jax.ShapeDtypeStruct((N, IN_CH), jnp.float32),
)


def _sc_layer1(yx_hbm, edge_hbm, agg_out, cnt_out,
               src_c, dst_c, src_t, dst_t, rows_v, zrow_v, cnt_v, y1_sh, acc_sh):
    cid = lax.axis_index("c")
    sid = lax.axis_index("s")
    wid = cid * NS + sid

    @pl.when(sid < 15)
    def _():
        r0 = sid * ROWS_PER_TILE
        pltpu.sync_copy(yx_hbm.at[pl.ds(r0, ROWS_PER_TILE), pl.ds(0, HID)],
                        y1_sh.at[pl.ds(r0, ROWS_PER_TILE), :])

    @pl.when(sid == 15)
    def _():
        r0 = 15 * ROWS_PER_TILE
        pltpu.sync_copy(yx_hbm.at[pl.ds(r0, LAST_ROWS), pl.ds(0, HID)],
                        y1_sh.at[pl.ds(r0, LAST_ROWS), :])

    zeros16 = jnp.zeros((16,), jnp.float32)

    def zb(i, carry):
        zrow_v[i, :] = zeros16
        cnt_v[i, :] = zeros16
        return carry

    lax.fori_loop(0, ROWS_PER_TILE, zb, 0)
    pltpu.sync_copy(zrow_v, acc_sh.at[pl.ds(sid * ROWS_PER_TILE, ROWS_PER_TILE), :])

    plsc.subcore_barrier()

    ones16 = jnp.full((16,), 1.0, jnp.float32)

    def do_chunk(idx_ref_s, idx_ref_d, rows_slice, size, base):
        pltpu.sync_copy(edge_hbm.at[0, pl.ds(base, size)], idx_ref_s)
        pltpu.sync_copy(edge_hbm.at[1, pl.ds(base, size)], idx_ref_d)
        pltpu.sync_copy(y1_sh.at[idx_ref_s], rows_slice)
        pltpu.sync_copy(rows_slice, acc_sh.at[idx_ref_d], add=True)

        def cnt_body(i, c2):
            d16 = idx_ref_d[pl.ds(i * 16, 16)]
            plsc.addupdate_scatter(cnt_v, [d16 >> 4, d16 & 15], ones16)
            return c2

        lax.fori_loop(0, size // 16, cnt_body, 0)

    def chunk(j, carry):
        do_chunk(src_c, dst_c, rows_v, CHUNK, wid * EPW + j * CHUNK)
        return carry

    lax.fori_loop(0, NFULL, chunk, 0)
    do_chunk(src_t, dst_t, rows_v.at[pl.ds(0, TAIL), :], TAIL,
             wid * EPW + NFULL * CHUNK)

    plsc.subcore_barrier()
    pltpu.sync_copy(acc_sh.at[pl.ds(sid * ROWS_PER_TILE, ROWS_PER_TILE), :],
                    agg_out.at[cid, pl.ds(sid * ROWS_PER_TILE, ROWS_PER_TILE), :])
    pltpu.sync_copy(cnt_v, cnt_out.at[wid])


_layer1_call = pl.kernel(
    _sc_layer1,
    out_type=(jax.ShapeDtypeStruct((NC, NPAD, HID), jnp.float32),
              jax.ShapeDtypeStruct((NW, ROWS_PER_TILE, HID), jnp.float32)),
    mesh=plsc.VectorSubcoreMesh(core_axis_name="c", subcore_axis_name="s",
                                num_cores=NC, num_subcores=NS),
    compiler_params=pltpu.CompilerParams(needs_layout_passes=False, use_tc_tiling_on_sc=False),
    scratch_types=[
        pltpu.VMEM((CHUNK,), jnp.int32),
        pltpu.VMEM((CHUNK,), jnp.int32),
        pltpu.VMEM((TAIL,), jnp.int32),
        pltpu.VMEM((TAIL,), jnp.int32),
        pltpu.VMEM((CHUNK, HID), jnp.float32),
        pltpu.VMEM((ROWS_PER_TILE, HID), jnp.float32),
        pltpu.VMEM((ROWS_PER_TILE, HID), jnp.float32),
        pltpu.VMEM_SHARED((N, HID), jnp.float32),
        pltpu.VMEM_SHARED((NPAD, HID), jnp.float32),
    ],
)


def _sc_mid2(agg_hbm, cntp_hbm, yx_hbm, b1_hbm, w2l_hbm, w2r_hbm, edge_hbm,
             accp_out, base2_out, c_out,
             a0_v, a1_v, xr_v, call_v, cnt_v, y2s_v, b2s_v, cs_v,
             b1_v, w2l_v, w2r_v, y2_v, srcf_v, dstf_v, acc_v, y2_sh):
    cid = lax.axis_index("c")
    sid = lax.axis_index("s")
    wid = cid * NS + sid

    pltpu.sync_copy(b1_hbm, b1_v)
    pltpu.sync_copy(w2l_hbm, w2l_v)
    pltpu.sync_copy(w2r_hbm, w2r_v)
    pltpu.sync_copy(edge_hbm.at[0, pl.ds(wid * EPW, EPW)], srcf_v)
    pltpu.sync_copy(edge_hbm.at[1, pl.ds(wid * EPW, EPW)], dstf_v)

    iota16 = lax.iota(jnp.int32, 16)
    zeros16 = jnp.zeros((16,), jnp.float32)

    def epilogue(ngrp):
        nrow = ngrp * 16
        row0 = sid * ROWS_PER_TILE
        pltpu.sync_copy(agg_hbm.at[0, pl.ds(row0, nrow), :],
                        a0_v.at[pl.ds(0, nrow), :])
        pltpu.sync_copy(agg_hbm.at[1, pl.ds(row0, nrow), :],
                        a1_v.at[pl.ds(0, nrow), :])
        pltpu.sync_copy(yx_hbm.at[pl.ds(row0, nrow), pl.ds(HID, HID)],
                        xr_v.at[pl.ds(0, nrow), :])
        pltpu.sync_copy(cntp_hbm.at[:, pl.ds(sid * NGRP, ngrp), :],
                        call_v.at[:, pl.ds(0, ngrp), :])

        def csum(g, carry):
            acc = call_v[0, g, :]
            for w in range(1, NW):
                acc = acc + call_v[w, g, :]
            cnt_v[g, :] = acc
            return carry

        lax.fori_loop(0, ngrp, csum, 0)

        b1vec = b1_v[...]
        w2lvec = w2l_v[...]
        w2rvec = w2r_v[...]

        def grp(g, carry):
            c16 = jnp.maximum(cnt_v[g, :], 1.0)
            rc = 1.0 / c16
            rows16 = g * 16 + iota16
            y2acc = zeros16
            b2acc = zeros16
            for k in range(HID):
                cols16 = jnp.full((16,), k, jnp.int32)
                a0g = plsc.load_gather(a0_v, [rows16, cols16])
                a1g = plsc.load_gather(a1_v, [rows16, cols16])
                xrg = plsc.load_gather(xr_v, [rows16, cols16])
                h16 = jnp.maximum((a0g + a1g) * rc + xrg + b1vec[k], 0.0)
                y2acc = y2acc + h16 * w2lvec[k]
                b2acc = b2acc + h16 * w2rvec[k]
            y2s_v[pl.ds(g * 16, 16)] = y2acc
            b2s_v[pl.ds(g * 16, 16)] = b2acc
            cs_v[pl.ds(g * 16, 16)] = c16
            return carry

        lax.fori_loop(0, ngrp, grp, 0)
        pltpu.sync_copy(y2s_v.at[pl.ds(0, nrow)], y2_sh.at[pl.ds(row0, nrow)])

        @pl.when(cid == 0)
        def _():
            pltpu.sync_copy(b2s_v.at[pl.ds(0, nrow)],
                            base2_out.at[pl.ds(row0, nrow)])
            pltpu.sync_copy(cs_v.at[pl.ds(0, nrow)],
                            c_out.at[pl.ds(row0, nrow)])

    @pl.when(sid < 15)
    def _():
        epilogue(NGRP)

    @pl.when(sid == 15)
    def _():
        epilogue(LAST_GRP)

    def zb(i, carry):
        acc_v[pl.ds(i * 16, 16)] = zeros16
        return carry

    lax.fori_loop(0, N // 16, zb, 0)

    plsc.subcore_barrier()
    pltpu.sync_copy(y2_sh, y2_v)

    def step(i, carry):
        s16 = srcf_v[pl.ds(i * 16, 16)]
        d16 = dstf_v[pl.ds(i * 16, 16)]
        vals = plsc.load_gather(y2_v, [s16])
        plsc.addupdate_scatter(acc_v, [d16], vals)
        return carry

    lax.fori_loop(0, EPW // 16, step, 0)

    pltpu.sync_copy(acc_v.at[pl.ds(0, N)], accp_out.at[pl.ds(wid * N, N)])


_mid2_call = pl.kernel(
    _sc_mid2,
    out_type=(jax.ShapeDtypeStruct((NW * N,), jnp.float32),
              jax.ShapeDtypeStruct((NPAD,), jnp.float32),
              jax.ShapeDtypeStruct((NPAD,), jnp.float32)),
    mesh=plsc.VectorSubcoreMesh(core_axis_name="c", subcore_axis_name="s",
                                num_cores=NC, num_subcores=NS),
    compiler_params=pltpu.CompilerParams(needs_layout_passes=False, use_tc_tiling_on_sc=False),
    scratch_types=[
        pltpu.VMEM((ROWS_PER_TILE, HID), jnp.float32),
        pltpu.VMEM((ROWS_PER_TILE, HID), jnp.float32),
        pltpu.VMEM((ROWS_PER_TILE, HID), jnp.float32),
        pltpu.VMEM((NW, NGRP, HID), jnp.float32),
        pltpu.VMEM((NGRP, HID), jnp.float32),
        pltpu.VMEM((ROWS_PER_TILE,), jnp.float32),
        pltpu.VMEM((ROWS_PER_TILE,), jnp.float32),
        pltpu.VMEM((ROWS_PER_TILE,), jnp.float32),
        pltpu.VMEM((HID,), jnp.float32),
        pltpu.VMEM((HID,), jnp.float32),
        pltpu.VMEM((HID,), jnp.float32),
        pltpu.VMEM((NPAD,), jnp.float32),
        pltpu.VMEM((EPW,), jnp.int32),
        pltpu.VMEM((EPW,), jnp.int32),
        pltpu.VMEM((N,), jnp.float32),
        pltpu.VMEM_SHARED((NPAD,), jnp.float32),
    ],
)


def _tc_final(aggp_ref, c_ref, base2_ref, b2_ref, out_ref):
    s = aggp_ref[pl.ds(0, N)]
    for w in range(1, NW):
        s = s + aggp_ref[pl.ds(w * N, N)]
    out_ref[...] = (s / c_ref[pl.ds(0, N)] + base2_ref[pl.ds(0, N)]
                    + b2_ref[...][0])


_final_call = pl.pallas_call(
    _tc_final,
    out_shape=jax.ShapeDtypeStruct((N,), jnp.float32),
)


def kernel(x, edge_index, W1_l, W1_r, b1, W2_l, W2_r, b2):
    ei = edge_index.astype(jnp.int32)
    wcat = jnp.zeros((IN_CH, IN_CH), jnp.float32)
    wcat = wcat.at[:, :HID].set(W1_l).at[:, HID:2 * HID].set(W1_r)
    yx = _transform_call(x, wcat)
    agg_p, cnt_p = _layer1_call(yx, ei)
    accp, base2, c = _mid2_call(agg_p, cnt_p, yx, b1, W2_l[:, 0], W2_r[:, 0], ei)
    return _final_call(accp, c, base2, b2)


# speedup vs baseline: 1.4376x; 1.1255x over previous
import functools

import jax
import jax.numpy as jnp
from jax import lax
from jax.experimental import pallas as pl
from jax.experimental.pallas import tpu as pltpu
from jax.experimental.pallas import tpu_sc as plsc

N = 10000
E = 320000
IN_CH = 128
HID = 16

NC, NS = 2, 16
NW = NC * NS
EPW = E // NW

CHUNK = 2048
NFULL = 4
TAIL = EPW - NFULL * CHUNK

NPAD = 10240
ROWS_PER_TILE = NPAD // NS
NGRP = ROWS_PER_TILE // 16
LAST_ROWS = N - 15 * ROWS_PER_TILE
LAST_GRP = LAST_ROWS // 16


def _tc_transform(x_ref, wcat_ref, yx_ref):
    yx_ref[...] = lax.dot(x_ref[...], wcat_ref[...],
                          precision=lax.Precision.HIGHEST,
                          preferred_element_type=jnp.float32)


_transform_call = pl.pallas_call(
    _tc_transform,
    out_shape=jax.ShapeDtypeStruct((N, IN_CH), jnp.float32),
)


def _sc_layer1(yx_hbm, edge_hbm, agg_out, cnt_out,
               src_a, dst_a, src_b, dst_b, src_t, dst_t, rows_a, rows_b,
               zrow_v, cnt_v, sem_i, sem_s, y1_sh, acc_sh):
    cid = lax.axis_index("c")
    sid = lax.axis_index("s")
    wid = cid * NS + sid

    @pl.when(sid < 15)
    def _():
        r0 = sid * ROWS_PER_TILE
        pltpu.sync_copy(yx_hbm.at[pl.ds(r0, ROWS_PER_TILE), pl.ds(0, HID)],
                        y1_sh.at[pl.ds(r0, ROWS_PER_TILE), :])

    @pl.when(sid == 15)
    def _():
        r0 = 15 * ROWS_PER_TILE
        pltpu.sync_copy(yx_hbm.at[pl.ds(r0, LAST_ROWS), pl.ds(0, HID)],
                        y1_sh.at[pl.ds(r0, LAST_ROWS), :])

    zeros16 = jnp.zeros((16,), jnp.float32)

    def zb(i, carry):
        zrow_v[i, :] = zeros16
        cnt_v[i, :] = zeros16
        return carry

    lax.fori_loop(0, ROWS_PER_TILE, zb, 0)
    pltpu.sync_copy(zrow_v, acc_sh.at[pl.ds(sid * ROWS_PER_TILE, ROWS_PER_TILE), :])

    plsc.subcore_barrier()

    ones16 = jnp.full((16,), 1.0, jnp.float32)

    idx_sets = [(src_a, dst_a), (src_b, dst_b)]
    rows_sets = [rows_a, rows_b]
    sizes = [CHUNK] * NFULL + [TAIL]

    def idx_refs(j):
        if j == NFULL:
            return src_t, dst_t
        return idx_sets[j % 2]

    def rows_ref(j):
        r = rows_sets[j % 2]
        if sizes[j] == CHUNK:
            return r
        return r.at[pl.ds(0, sizes[j]), :]

    s0, d0 = idx_refs(0)
    pltpu.sync_copy(edge_hbm.at[0, pl.ds(wid * EPW, CHUNK)], s0)
    pltpu.sync_copy(edge_hbm.at[1, pl.ds(wid * EPW, CHUNK)], d0)

    pending = [None, None]
    for j in range(NFULL + 1):
        cur = j % 2
        nxt = (j + 1) % 2
        if pending[nxt] is not None:
            pending[nxt].wait()
            pending[nxt] = None
        dpre = []
        if j < NFULL:
            base = wid * EPW + (j + 1) * CHUNK
            sn, dn = idx_refs(j + 1)
            dpre.append(pltpu.async_copy(
                edge_hbm.at[0, pl.ds(base, sizes[j + 1])], sn, sem_i))
            dpre.append(pltpu.async_copy(
                edge_hbm.at[1, pl.ds(base, sizes[j + 1])], dn, sem_i))
        sj, dj = idx_refs(j)
        rj = rows_ref(j)
        pltpu.sync_copy(y1_sh.at[sj], rj)
        pending[cur] = pltpu.async_copy(rj, acc_sh.at[dj], sem_s, add=True)

        @plsc.parallel_loop(0, sizes[j] // 16, unroll=4)
        def _(i, _dj=dj):
            d16 = _dj[pl.ds(i * 16, 16)]
            plsc.addupdate_scatter(cnt_v, [d16 >> 4, d16 & 15], ones16)

        for d in dpre:
            d.wait()

    for p in pending:
        if p is not None:
            p.wait()

    plsc.subcore_barrier()
    pltpu.sync_copy(acc_sh.at[pl.ds(sid * ROWS_PER_TILE, ROWS_PER_TILE), :],
                    agg_out.at[cid, pl.ds(sid * ROWS_PER_TILE, ROWS_PER_TILE), :])
    pltpu.sync_copy(cnt_v, cnt_out.at[wid])


_layer1_call = pl.kernel(
    _sc_layer1,
    out_type=(jax.ShapeDtypeStruct((NC, NPAD, HID), jnp.float32),
              jax.ShapeDtypeStruct((NW, ROWS_PER_TILE, HID), jnp.float32)),
    mesh=plsc.VectorSubcoreMesh(core_axis_name="c", subcore_axis_name="s",
                                num_cores=NC, num_subcores=NS),
    compiler_params=pltpu.CompilerParams(needs_layout_passes=False, use_tc_tiling_on_sc=False),
    scratch_types=[
        pltpu.VMEM((CHUNK,), jnp.int32),
        pltpu.VMEM((CHUNK,), jnp.int32),
        pltpu.VMEM((CHUNK,), jnp.int32),
        pltpu.VMEM((CHUNK,), jnp.int32),
        pltpu.VMEM((TAIL,), jnp.int32),
        pltpu.VMEM((TAIL,), jnp.int32),
        pltpu.VMEM((CHUNK, HID), jnp.float32),
        pltpu.VMEM((CHUNK, HID), jnp.float32),
        pltpu.VMEM((ROWS_PER_TILE, HID), jnp.float32),
        pltpu.VMEM((ROWS_PER_TILE, HID), jnp.float32),
        pltpu.SemaphoreType.DMA,
        pltpu.SemaphoreType.DMA,
        pltpu.VMEM_SHARED((N, HID), jnp.float32),
        pltpu.VMEM_SHARED((NPAD, HID), jnp.float32),
    ],
)


def _sc_mid2(agg_hbm, cntp_hbm, yx_hbm, b1_hbm, w2l_hbm, w2r_hbm, edge_hbm,
             accp_out, base2_out, c_out,
             a0_v, a1_v, xr_v, call_v, cnt_v, y2s_v, b2s_v, cs_v,
             b1_v, w2l_v, w2r_v, y2_v, srcf_v, dstf_v, acc_v, y2_sh):
    cid = lax.axis_index("c")
    sid = lax.axis_index("s")
    wid = cid * NS + sid

    pltpu.sync_copy(b1_hbm, b1_v)
    pltpu.sync_copy(w2l_hbm, w2l_v)
    pltpu.sync_copy(w2r_hbm, w2r_v)
    pltpu.sync_copy(edge_hbm.at[0, pl.ds(wid * EPW, EPW)], srcf_v)
    pltpu.sync_copy(edge_hbm.at[1, pl.ds(wid * EPW, EPW)], dstf_v)

    iota16 = lax.iota(jnp.int32, 16)
    zeros16 = jnp.zeros((16,), jnp.float32)

    def epilogue(ngrp):
        nrow = ngrp * 16
        row0 = sid * ROWS_PER_TILE
        pltpu.sync_copy(agg_hbm.at[0, pl.ds(row0, nrow), :],
                        a0_v.at[pl.ds(0, nrow), :])
        pltpu.sync_copy(agg_hbm.at[1, pl.ds(row0, nrow), :],
                        a1_v.at[pl.ds(0, nrow), :])
        pltpu.sync_copy(yx_hbm.at[pl.ds(row0, nrow), pl.ds(HID, HID)],
                        xr_v.at[pl.ds(0, nrow), :])
        pltpu.sync_copy(cntp_hbm.at[:, pl.ds(sid * NGRP, ngrp), :],
                        call_v.at[:, pl.ds(0, ngrp), :])

        @plsc.parallel_loop(0, ngrp, unroll=2)
        def _(g):
            acc = call_v[0, g, :]
            for w in range(1, NW):
                acc = acc + call_v[w, g, :]
            cnt_v[g, :] = acc

        b1vec = b1_v[...]
        w2lvec = w2l_v[...]
        w2rvec = w2r_v[...]

        @plsc.parallel_loop(0, ngrp)
        def grp(g):
            c16 = jnp.maximum(cnt_v[g, :], 1.0)
            rc = 1.0 / c16
            rows16 = g * 16 + iota16
            y2acc = zeros16
            b2acc = zeros16
            for k in range(HID):
                cols16 = jnp.full((16,), k, jnp.int32)
                a0g = plsc.load_gather(a0_v, [rows16, cols16])
                a1g = plsc.load_gather(a1_v, [rows16, cols16])
                xrg = plsc.load_gather(xr_v, [rows16, cols16])
                h16 = jnp.maximum((a0g + a1g) * rc + xrg + b1vec[k], 0.0)
                y2acc = y2acc + h16 * w2lvec[k]
                b2acc = b2acc + h16 * w2rvec[k]
            y2s_v[pl.ds(g * 16, 16)] = y2acc
            b2s_v[pl.ds(g * 16, 16)] = b2acc
            cs_v[pl.ds(g * 16, 16)] = c16
        pltpu.sync_copy(y2s_v.at[pl.ds(0, nrow)], y2_sh.at[pl.ds(row0, nrow)])

        @pl.when(cid == 0)
        def _():
            pltpu.sync_copy(b2s_v.at[pl.ds(0, nrow)],
                            base2_out.at[pl.ds(row0, nrow)])
            pltpu.sync_copy(cs_v.at[pl.ds(0, nrow)],
                            c_out.at[pl.ds(row0, nrow)])

    @pl.when(sid < 15)
    def _():
        epilogue(NGRP)

    @pl.when(sid == 15)
    def _():
        epilogue(LAST_GRP)

    @plsc.parallel_loop(0, N // 16, unroll=8)
    def _(i):
        acc_v[pl.ds(i * 16, 16)] = zeros16

    plsc.subcore_barrier()
    pltpu.sync_copy(y2_sh, y2_v)

    @plsc.parallel_loop(0, EPW // 16, unroll=5)
    def _(i):
        s16 = srcf_v[pl.ds(i * 16, 16)]
        d16 = dstf_v[pl.ds(i * 16, 16)]
        vals = plsc.load_gather(y2_v, [s16])
        plsc.addupdate_scatter(acc_v, [d16], vals)

    pltpu.sync_copy(acc_v.at[pl.ds(0, N)], accp_out.at[pl.ds(wid * N, N)])


_mid2_call = pl.kernel(
    _sc_mid2,
    out_type=(jax.ShapeDtypeStruct((NW * N,), jnp.float32),
              jax.ShapeDtypeStruct((NPAD,), jnp.float32),
              jax.ShapeDtypeStruct((NPAD,), jnp.float32)),
    mesh=plsc.VectorSubcoreMesh(core_axis_name="c", subcore_axis_name="s",
                                num_cores=NC, num_subcores=NS),
    compiler_params=pltpu.CompilerParams(needs_layout_passes=False, use_tc_tiling_on_sc=False),
    scratch_types=[
        pltpu.VMEM((ROWS_PER_TILE, HID), jnp.float32),
        pltpu.VMEM((ROWS_PER_TILE, HID), jnp.float32),
        pltpu.VMEM((ROWS_PER_TILE, HID), jnp.float32),
        pltpu.VMEM((NW, NGRP, HID), jnp.float32),
        pltpu.VMEM((NGRP, HID), jnp.float32),
        pltpu.VMEM((ROWS_PER_TILE,), jnp.float32),
        pltpu.VMEM((ROWS_PER_TILE,), jnp.float32),
        pltpu.VMEM((ROWS_PER_TILE,), jnp.float32),
        pltpu.VMEM((HID,), jnp.float32),
        pltpu.VMEM((HID,), jnp.float32),
        pltpu.VMEM((HID,), jnp.float32),
        pltpu.VMEM((NPAD,), jnp.float32),
        pltpu.VMEM((EPW,), jnp.int32),
        pltpu.VMEM((EPW,), jnp.int32),
        pltpu.VMEM((N,), jnp.float32),
        pltpu.VMEM_SHARED((NPAD,), jnp.float32),
    ],
)


def _tc_final(aggp_ref, c_ref, base2_ref, b2_ref, out_ref):
    s = aggp_ref[pl.ds(0, N)]
    for w in range(1, NW):
        s = s + aggp_ref[pl.ds(w * N, N)]
    out_ref[...] = (s / c_ref[pl.ds(0, N)] + base2_ref[pl.ds(0, N)]
                    + b2_ref[...][0])


_final_call = pl.pallas_call(
    _tc_final,
    out_shape=jax.ShapeDtypeStruct((N,), jnp.float32),
)


def kernel(x, edge_index, W1_l, W1_r, b1, W2_l, W2_r, b2):
    ei = edge_index.astype(jnp.int32)
    wcat = jnp.zeros((IN_CH, IN_CH), jnp.float32)
    wcat = wcat.at[:, :HID].set(W1_l).at[:, HID:2 * HID].set(W1_r)
    yx = _transform_call(x, wcat)
    agg_p, cnt_p = _layer1_call(yx, ei)
    accp, base2, c = _mid2_call(agg_p, cnt_p, yx, b1, W2_l[:, 0], W2_r[:, 0], ei)
    return _final_call(accp, c, base2, b2)


# speedup vs baseline: 1.5344x; 1.0673x over previous
import functools

import jax
import jax.numpy as jnp
from jax import lax
from jax.experimental import pallas as pl
from jax.experimental.pallas import tpu as pltpu
from jax.experimental.pallas import tpu_sc as plsc

N = 10000
E = 320000
IN_CH = 128
HID = 16

NC, NS = 2, 16
NW = NC * NS
EPW = E // NW

CHUNK = 2048
NFULL = 4
TAIL = EPW - NFULL * CHUNK

NPAD = 10240
ROWS_PER_TILE = NPAD // NS
NGRP = ROWS_PER_TILE // 16
LAST_ROWS = N - 15 * ROWS_PER_TILE
LAST_GRP = LAST_ROWS // 16


def _tc_transform(x_ref, wl_ref, wr_ref, yx_ref):
    wcat = jnp.concatenate([wl_ref[...], wr_ref[...]], axis=1)
    y = lax.dot(x_ref[...], wcat, precision=lax.Precision.HIGHEST,
                preferred_element_type=jnp.float32)
    yx_ref[:, 0:2 * HID] = y


_transform_call = pl.pallas_call(
    _tc_transform,
    out_shape=jax.ShapeDtypeStruct((N, IN_CH), jnp.float32),
)


def _sc_layer1(yx_hbm, edge_hbm, agg_out, cnt_out,
               src_a, dst_a, src_b, dst_b, src_t, dst_t, rows_a, rows_b,
               zrow_v, cnt_v, sem_i, sem_s, y1_sh, acc_sh):
    cid = lax.axis_index("c")
    sid = lax.axis_index("s")
    wid = cid * NS + sid

    @pl.when(sid < 15)
    def _():
        r0 = sid * ROWS_PER_TILE
        pltpu.sync_copy(yx_hbm.at[pl.ds(r0, ROWS_PER_TILE), pl.ds(0, HID)],
                        y1_sh.at[pl.ds(r0, ROWS_PER_TILE), :])

    @pl.when(sid == 15)
    def _():
        r0 = 15 * ROWS_PER_TILE
        pltpu.sync_copy(yx_hbm.at[pl.ds(r0, LAST_ROWS), pl.ds(0, HID)],
                        y1_sh.at[pl.ds(r0, LAST_ROWS), :])

    zeros16 = jnp.zeros((16,), jnp.float32)

    def zb(i, carry):
        zrow_v[i, :] = zeros16
        cnt_v[i, :] = zeros16
        return carry

    lax.fori_loop(0, ROWS_PER_TILE, zb, 0)
    pltpu.sync_copy(zrow_v, acc_sh.at[pl.ds(sid * ROWS_PER_TILE, ROWS_PER_TILE), :])

    plsc.subcore_barrier()

    ones16 = jnp.full((16,), 1.0, jnp.float32)

    idx_sets = [(src_a, dst_a), (src_b, dst_b)]
    rows_sets = [rows_a, rows_b]
    sizes = [CHUNK] * NFULL + [TAIL]

    def idx_refs(j):
        if j == NFULL:
            return src_t, dst_t
        return idx_sets[j % 2]

    def rows_ref(j):
        r = rows_sets[j % 2]
        if sizes[j] == CHUNK:
            return r
        return r.at[pl.ds(0, sizes[j]), :]

    s0, d0 = idx_refs(0)
    pltpu.sync_copy(edge_hbm.at[0, pl.ds(wid * EPW, CHUNK)], s0)
    pltpu.sync_copy(edge_hbm.at[1, pl.ds(wid * EPW, CHUNK)], d0)

    pending = [None, None]
    for j in range(NFULL + 1):
        cur = j % 2
        nxt = (j + 1) % 2
        if pending[nxt] is not None:
            pending[nxt].wait()
            pending[nxt] = None
        dpre = []
        if j < NFULL:
            base = wid * EPW + (j + 1) * CHUNK
            sn, dn = idx_refs(j + 1)
            dpre.append(pltpu.async_copy(
                edge_hbm.at[0, pl.ds(base, sizes[j + 1])], sn, sem_i))
            dpre.append(pltpu.async_copy(
                edge_hbm.at[1, pl.ds(base, sizes[j + 1])], dn, sem_i))
        sj, dj = idx_refs(j)
        rj = rows_ref(j)
        pltpu.sync_copy(y1_sh.at[sj], rj)
        pending[cur] = pltpu.async_copy(rj, acc_sh.at[dj], sem_s, add=True)

        @plsc.parallel_loop(0, sizes[j] // 16, unroll=4)
        def _(i, _dj=dj):
            d16 = _dj[pl.ds(i * 16, 16)]
            plsc.addupdate_scatter(cnt_v, [d16 >> 4, d16 & 15], ones16)

        for d in dpre:
            d.wait()

    for p in pending:
        if p is not None:
            p.wait()

    plsc.subcore_barrier()
    pltpu.sync_copy(acc_sh.at[pl.ds(sid * ROWS_PER_TILE, ROWS_PER_TILE), :],
                    agg_out.at[cid, pl.ds(sid * ROWS_PER_TILE, ROWS_PER_TILE), :])
    pltpu.sync_copy(cnt_v, cnt_out.at[wid])


_layer1_call = pl.kernel(
    _sc_layer1,
    out_type=(jax.ShapeDtypeStruct((NC, NPAD, HID), jnp.float32),
              jax.ShapeDtypeStruct((NW, ROWS_PER_TILE, HID), jnp.float32)),
    mesh=plsc.VectorSubcoreMesh(core_axis_name="c", subcore_axis_name="s",
                                num_cores=NC, num_subcores=NS),
    compiler_params=pltpu.CompilerParams(needs_layout_passes=False, use_tc_tiling_on_sc=False),
    scratch_types=[
        pltpu.VMEM((CHUNK,), jnp.int32),
        pltpu.VMEM((CHUNK,), jnp.int32),
        pltpu.VMEM((CHUNK,), jnp.int32),
        pltpu.VMEM((CHUNK,), jnp.int32),
        pltpu.VMEM((TAIL,), jnp.int32),
        pltpu.VMEM((TAIL,), jnp.int32),
        pltpu.VMEM((CHUNK, HID), jnp.float32),
        pltpu.VMEM((CHUNK, HID), jnp.float32),
        pltpu.VMEM((ROWS_PER_TILE, HID), jnp.float32),
        pltpu.VMEM((ROWS_PER_TILE, HID), jnp.float32),
        pltpu.SemaphoreType.DMA,
        pltpu.SemaphoreType.DMA,
        pltpu.VMEM_SHARED((N, HID), jnp.float32),
        pltpu.VMEM_SHARED((NPAD, HID), jnp.float32),
    ],
)


def _sc_mid2(agg_hbm, cntp_hbm, yx_hbm, b1_hbm, w2l_hbm, w2r_hbm, edge_hbm,
             accp_out, base2_out, c_out,
             a0_v, a1_v, xr_v, call_v, cnt_v, y2s_v, b2s_v, cs_v,
             b1_v, w2l_v, w2r_v, y2_v, srcf_v, dstf_v, acc_v, sem_e, sem_st, y2_sh):
    cid = lax.axis_index("c")
    sid = lax.axis_index("s")
    wid = cid * NS + sid

    pltpu.sync_copy(b1_hbm, b1_v)
    pltpu.sync_copy(w2l_hbm, w2l_v)
    pltpu.sync_copy(w2r_hbm, w2r_v)
    d_edges = [
        pltpu.async_copy(edge_hbm.at[0, pl.ds(wid * EPW, EPW)], srcf_v, sem_e),
        pltpu.async_copy(edge_hbm.at[1, pl.ds(wid * EPW, EPW)], dstf_v, sem_e),
    ]

    iota16 = lax.iota(jnp.int32, 16)
    zeros16 = jnp.zeros((16,), jnp.float32)

    def epilogue(ngrp):
        nrow = ngrp * 16
        row0 = sid * ROWS_PER_TILE
        d_st = [
            pltpu.async_copy(agg_hbm.at[0, pl.ds(row0, nrow), :],
                             a0_v.at[pl.ds(0, nrow), :], sem_st),
            pltpu.async_copy(agg_hbm.at[1, pl.ds(row0, nrow), :],
                             a1_v.at[pl.ds(0, nrow), :], sem_st),
            pltpu.async_copy(yx_hbm.at[pl.ds(row0, nrow), pl.ds(HID, HID)],
                             xr_v.at[pl.ds(0, nrow), :], sem_st),
            pltpu.async_copy(cntp_hbm.at[:, pl.ds(sid * NGRP, ngrp), :],
                             call_v.at[:, pl.ds(0, ngrp), :], sem_st),
        ]
        for d in d_st:
            d.wait()

        @plsc.parallel_loop(0, ngrp, unroll=2)
        def _(g):
            acc = call_v[0, g, :]
            for w in range(1, NW):
                acc = acc + call_v[w, g, :]
            cnt_v[g, :] = acc

        b1vec = b1_v[...]
        w2lvec = w2l_v[...]
        w2rvec = w2r_v[...]

        @plsc.parallel_loop(0, ngrp)
        def grp(g):
            c16 = jnp.maximum(cnt_v[g, :], 1.0)
            rc = 1.0 / c16
            rows16 = g * 16 + iota16
            y2acc = zeros16
            b2acc = zeros16
            for k in range(HID):
                cols16 = jnp.full((16,), k, jnp.int32)
                a0g = plsc.load_gather(a0_v, [rows16, cols16])
                a1g = plsc.load_gather(a1_v, [rows16, cols16])
                xrg = plsc.load_gather(xr_v, [rows16, cols16])
                h16 = jnp.maximum((a0g + a1g) * rc + xrg + b1vec[k], 0.0)
                y2acc = y2acc + h16 * w2lvec[k]
                b2acc = b2acc + h16 * w2rvec[k]
            y2s_v[pl.ds(g * 16, 16)] = y2acc
            b2s_v[pl.ds(g * 16, 16)] = b2acc
            cs_v[pl.ds(g * 16, 16)] = c16
        pltpu.sync_copy(y2s_v.at[pl.ds(0, nrow)], y2_sh.at[pl.ds(row0, nrow)])

        @pl.when(cid == 0)
        def _():
            pltpu.sync_copy(b2s_v.at[pl.ds(0, nrow)],
                            base2_out.at[pl.ds(row0, nrow)])
            pltpu.sync_copy(cs_v.at[pl.ds(0, nrow)],
                            c_out.at[pl.ds(row0, nrow)])

    @pl.when(sid < 15)
    def _():
        epilogue(NGRP)

    @pl.when(sid == 15)
    def _():
        epilogue(LAST_GRP)

    @plsc.parallel_loop(0, N // 16, unroll=8)
    def _(i):
        acc_v[pl.ds(i * 16, 16)] = zeros16

    for d in d_edges:
        d.wait()
    plsc.subcore_barrier()
    pltpu.sync_copy(y2_sh, y2_v)

    @plsc.parallel_loop(0, EPW // 16, unroll=5)
    def _(i):
        s16 = srcf_v[pl.ds(i * 16, 16)]
        d16 = dstf_v[pl.ds(i * 16, 16)]
        vals = plsc.load_gather(y2_v, [s16])
        plsc.addupdate_scatter(acc_v, [d16], vals)

    pltpu.sync_copy(acc_v.at[pl.ds(0, N)], accp_out.at[pl.ds(wid * N, N)])


_mid2_call = pl.kernel(
    _sc_mid2,
    out_type=(jax.ShapeDtypeStruct((NW * N,), jnp.float32),
              jax.ShapeDtypeStruct((NPAD,), jnp.float32),
              jax.ShapeDtypeStruct((NPAD,), jnp.float32)),
    mesh=plsc.VectorSubcoreMesh(core_axis_name="c", subcore_axis_name="s",
                                num_cores=NC, num_subcores=NS),
    compiler_params=pltpu.CompilerParams(needs_layout_passes=False, use_tc_tiling_on_sc=False),
    scratch_types=[
        pltpu.VMEM((ROWS_PER_TILE, HID), jnp.float32),
        pltpu.VMEM((ROWS_PER_TILE, HID), jnp.float32),
        pltpu.VMEM((ROWS_PER_TILE, HID), jnp.float32),
        pltpu.VMEM((NW, NGRP, HID), jnp.float32),
        pltpu.VMEM((NGRP, HID), jnp.float32),
        pltpu.VMEM((ROWS_PER_TILE,), jnp.float32),
        pltpu.VMEM((ROWS_PER_TILE,), jnp.float32),
        pltpu.VMEM((ROWS_PER_TILE,), jnp.float32),
        pltpu.VMEM((HID,), jnp.float32),
        pltpu.VMEM((HID,), jnp.float32),
        pltpu.VMEM((HID,), jnp.float32),
        pltpu.VMEM((NPAD,), jnp.float32),
        pltpu.VMEM((EPW,), jnp.int32),
        pltpu.VMEM((EPW,), jnp.int32),
        pltpu.VMEM((N,), jnp.float32),
        pltpu.SemaphoreType.DMA,
        pltpu.SemaphoreType.DMA,
        pltpu.VMEM_SHARED((NPAD,), jnp.float32),
    ],
)


def _tc_final(aggp_ref, c_ref, base2_ref, b2_ref, out_ref):
    s = aggp_ref[pl.ds(0, N)]
    for w in range(1, NW):
        s = s + aggp_ref[pl.ds(w * N, N)]
    out_ref[...] = (s / c_ref[pl.ds(0, N)] + base2_ref[pl.ds(0, N)]
                    + b2_ref[...][0])


_final_call = pl.pallas_call(
    _tc_final,
    out_shape=jax.ShapeDtypeStruct((N,), jnp.float32),
)


def kernel(x, edge_index, W1_l, W1_r, b1, W2_l, W2_r, b2):
    ei = edge_index.astype(jnp.int32)
    yx = _transform_call(x, W1_l, W1_r)
    agg_p, cnt_p = _layer1_call(yx, ei)
    accp, base2, c = _mid2_call(agg_p, cnt_p, yx, b1, W2_l[:, 0], W2_r[:, 0], ei)
    return _final_call(accp, c, base2, b2)
